# baseline (speedup 1.0000x reference)
import jax
import jax.numpy as jnp
from jax import lax
from jax.experimental import pallas as pl
from jax.experimental.pallas import tpu as pltpu
from jax.experimental.pallas import tpu_sc as plsc

B = 16384
F = 128

_NC = 2
_NS = 16
_NW = _NC * _NS
_IDX_W = 128
_NROWS = B // _IDX_W
_HROWS = _NROWS // 2
_ROWS_PW = _HROWS // _NW


def _sc_gather(ts_hbm, ids_hbm, out_hbm, idx_v, rows_v, sem):
    wid = lax.axis_index("s") * _NC + lax.axis_index("c")
    base = wid * _ROWS_PW
    pltpu.sync_copy(ids_hbm.at[pl.ds(base, _ROWS_PW)], idx_v)
    copies = [
        pltpu.async_copy(ts_hbm.at[idx_v.at[j]], rows_v.at[j], sem)
        for j in range(_ROWS_PW)
    ]
    for c in copies:
        c.wait()
    pltpu.sync_copy(rows_v, out_hbm.at[pl.ds(base, _ROWS_PW)])


def _gather_shifts(time_shifts, ids_half):
    mesh = plsc.VectorSubcoreMesh(core_axis_name="c", subcore_axis_name="s")
    fn = pl.kernel(
        _sc_gather,
        out_type=jax.ShapeDtypeStruct((_HROWS, _IDX_W), jnp.float32),
        mesh=mesh,
        scratch_types=[
            pltpu.VMEM((_ROWS_PW, _IDX_W), jnp.int32),
            pltpu.VMEM((_ROWS_PW, _IDX_W), jnp.float32),
            pltpu.SemaphoreType.DMA,
        ],
    )
    return fn(time_shifts, ids_half)


_R = 2048
_RC = _R // _IDX_W
_HBLOCKS = (B // 2) // _R


def _dense_body(t_ref, sh_ref, k_ref, x0_ref, o_ref):
    s = t_ref[...] + sh_ref[...]
    st = s.T
    hkv = 0.5 * jnp.exp(k_ref[...])
    hkx0 = hkv * x0_ref[...]
    for j in range(_RC):
        col = lax.slice(st, (0, j), (F, j + 1))
        o_ref[pl.ds(j * F, F), :] = 0.5 * jnp.tanh(hkv * col - hkx0) + 0.5


def _dense_alias_body(t_ref, sh_ref, k_ref, x0_ref, buf_ref, o_ref):
    del buf_ref
    _dense_body(t_ref, sh_ref, k_ref, x0_ref, o_ref)


def _dense_half0(t2d, sh0, k2, x02):
    return pl.pallas_call(
        _dense_body,
        grid=(_HBLOCKS,),
        in_specs=[
            pl.BlockSpec((_RC, _IDX_W), lambda i: (i, 0)),
            pl.BlockSpec((_RC, _IDX_W), lambda i: (i, 0)),
            pl.BlockSpec((1, F), lambda i: (0, 0)),
            pl.BlockSpec((1, F), lambda i: (0, 0)),
        ],
        out_specs=pl.BlockSpec((_R, F), lambda i: (i, 0)),
        out_shape=jax.ShapeDtypeStruct((B, F), jnp.float32),
    )(t2d, sh0, k2, x02)


def _dense_half1(t2d, sh1, k2, x02, buf):
    return pl.pallas_call(
        _dense_alias_body,
        grid=(_HBLOCKS,),
        in_specs=[
            pl.BlockSpec((_RC, _IDX_W), lambda i: (i + _HBLOCKS, 0)),
            pl.BlockSpec((_RC, _IDX_W), lambda i: (i, 0)),
            pl.BlockSpec((1, F), lambda i: (0, 0)),
            pl.BlockSpec((1, F), lambda i: (0, 0)),
            pl.BlockSpec((8, F), lambda i: (0, 0)),
        ],
        out_specs=pl.BlockSpec((_R, F), lambda i: (i + _HBLOCKS, 0)),
        out_shape=jax.ShapeDtypeStruct((B, F), jnp.float32),
        input_output_aliases={4: 0},
    )(t2d, sh1, k2, x02, buf)


def kernel(t, participant_ids, log_k_values, x0_values, time_shifts):
    ids2d = participant_ids.astype(jnp.int32).reshape(_NROWS, _IDX_W)
    t2d = t.reshape(_NROWS, _IDX_W)
    k2 = log_k_values.reshape(1, F)
    x02 = x0_values.reshape(1, F)
    sh0 = _gather_shifts(time_shifts, ids2d[:_HROWS])
    sh1 = _gather_shifts(time_shifts, ids2d[_HROWS:])
    buf = _dense_half0(t2d, sh0, k2, x02)
    return _dense_half1(t2d, sh1, k2, x02, buf)


# speedup vs baseline: 1.1298x; 1.1298x over previous
import jax
import jax.numpy as jnp
from jax import lax
from jax.experimental import pallas as pl
from jax.experimental.pallas import tpu as pltpu
from jax.experimental.pallas import tpu_sc as plsc

B = 16384
F = 128

_NC = 2
_NS = 16
_NW = _NC * _NS
_IDX_W = 128
_NROWS = B // _IDX_W
_ROWS_PW = _NROWS // _NW


def _sc_gather(ts_hbm, ids_hbm, out_hbm, idx_v, rows_v, sem):
    wid = lax.axis_index("s") * _NC + lax.axis_index("c")
    base = wid * _ROWS_PW
    pltpu.sync_copy(ids_hbm.at[pl.ds(base, _ROWS_PW)], idx_v)
    copies = [
        pltpu.async_copy(ts_hbm.at[idx_v.at[j]], rows_v.at[j], sem)
        for j in range(_ROWS_PW)
    ]
    for j, c in enumerate(copies):
        c.wait()
        pltpu.sync_copy(rows_v.at[j], out_hbm.at[base + j])


def _gather_shifts(time_shifts, ids2d):
    mesh = plsc.VectorSubcoreMesh(core_axis_name="c", subcore_axis_name="s")
    fn = pl.kernel(
        _sc_gather,
        out_type=jax.ShapeDtypeStruct((_NROWS, _IDX_W), jnp.float32),
        mesh=mesh,
        scratch_types=[
            pltpu.VMEM((_ROWS_PW, _IDX_W), jnp.int32),
            pltpu.VMEM((_ROWS_PW, _IDX_W), jnp.float32),
            pltpu.SemaphoreType.DMA,
        ],
    )
    return fn(time_shifts, ids2d)


_R = 4096
_RC = _R // _IDX_W


def _dense_body(t_ref, sh_ref, k_ref, x0_ref, o_ref):
    s = t_ref[...] + sh_ref[...]
    st = s.T
    hkv = 0.5 * jnp.exp(k_ref[...])
    hkx0 = hkv * x0_ref[...]
    for j in range(_RC):
        col = lax.slice(st, (0, j), (F, j + 1))
        o_ref[pl.ds(j * F, F), :] = 0.5 * jnp.tanh(hkv * col - hkx0) + 0.5


def _dense(t2d, sh2d, k2, x02):
    return pl.pallas_call(
        _dense_body,
        grid=(B // _R,),
        in_specs=[
            pl.BlockSpec((_RC, _IDX_W), lambda i: (i, 0)),
            pl.BlockSpec((_RC, _IDX_W), lambda i: (i, 0)),
            pl.BlockSpec((1, F), lambda i: (0, 0)),
            pl.BlockSpec((1, F), lambda i: (0, 0)),
        ],
        out_specs=pl.BlockSpec((_R, F), lambda i: (i, 0)),
        out_shape=jax.ShapeDtypeStruct((B, F), jnp.float32),
    )(t2d, sh2d, k2, x02)


def kernel(t, participant_ids, log_k_values, x0_values, time_shifts):
    ids2d = participant_ids.astype(jnp.int32).reshape(_NROWS, _IDX_W)
    shift2d = _gather_shifts(time_shifts, ids2d)
    return _dense(
        t.reshape(_NROWS, _IDX_W),
        shift2d,
        log_k_values.reshape(1, F),
        x0_values.reshape(1, F),
    )


# speedup vs baseline: 1.1405x; 1.0095x over previous
import jax
import jax.numpy as jnp
from jax import lax
from jax.experimental import pallas as pl
from jax.experimental.pallas import tpu as pltpu
from jax.experimental.pallas import tpu_sc as plsc

B = 16384
F = 128

_NC = 2
_NS = 16
_NW = _NC * _NS
_IDX_W = 128
_NROWS = B // _IDX_W
_ROWS_PW = _NROWS // _NW


def _sc_gather(ts_hbm, ids_hbm, out_hbm, idx_v, rows_v, sem):
    wid = lax.axis_index("s") * _NC + lax.axis_index("c")
    base = wid * _ROWS_PW
    pltpu.sync_copy(ids_hbm.at[pl.ds(base, _ROWS_PW)], idx_v)
    copies = [
        pltpu.async_copy(ts_hbm.at[idx_v.at[j]], rows_v.at[j], sem)
        for j in range(_ROWS_PW)
    ]
    for c in copies:
        c.wait()
    pltpu.sync_copy(rows_v, out_hbm.at[pl.ds(base, _ROWS_PW)])


def _gather_shifts(time_shifts, ids2d):
    mesh = plsc.VectorSubcoreMesh(core_axis_name="c", subcore_axis_name="s")
    fn = pl.kernel(
        _sc_gather,
        out_type=jax.ShapeDtypeStruct((_NROWS, _IDX_W), jnp.float32),
        mesh=mesh,
        scratch_types=[
            pltpu.VMEM((_ROWS_PW, _IDX_W), jnp.int32),
            pltpu.VMEM((_ROWS_PW, _IDX_W), jnp.float32),
            pltpu.SemaphoreType.DMA,
        ],
    )
    return fn(time_shifts, ids2d)


_R = 4096
_RC = _R // _IDX_W


def _dense_body(t_ref, sh_ref, k_ref, x0_ref, o_ref):
    s = t_ref[...] + sh_ref[...]
    st = s.T
    hkv = 0.5 * jnp.exp(k_ref[...])
    hkx0 = hkv * x0_ref[...]
    for j in range(_RC):
        col = lax.slice(st, (0, j), (F, j + 1))
        o_ref[pl.ds(j * F, F), :] = 0.5 * jnp.tanh(hkv * col - hkx0) + 0.5


def _dense(t2d, sh2d, k2, x02):
    return pl.pallas_call(
        _dense_body,
        grid=(B // _R,),
        in_specs=[
            pl.BlockSpec((_RC, _IDX_W), lambda i: (i, 0)),
            pl.BlockSpec((_RC, _IDX_W), lambda i: (i, 0)),
            pl.BlockSpec((1, F), lambda i: (0, 0)),
            pl.BlockSpec((1, F), lambda i: (0, 0)),
        ],
        out_specs=pl.BlockSpec((_R, F), lambda i: (i, 0)),
        out_shape=jax.ShapeDtypeStruct((B, F), jnp.float32),
    )(t2d, sh2d, k2, x02)


def kernel(t, participant_ids, log_k_values, x0_values, time_shifts):
    ids2d = participant_ids.astype(jnp.int32).reshape(_NROWS, _IDX_W)
    shift2d = _gather_shifts(time_shifts, ids2d)
    return _dense(
        t.reshape(_NROWS, _IDX_W),
        shift2d,
        log_k_values.reshape(1, F),
        x0_values.reshape(1, F),
    )


# speedup vs baseline: 1.1681x; 1.0242x over previous
import jax
import jax.numpy as jnp
from jax import lax
from jax.experimental import pallas as pl
from jax.experimental.pallas import tpu as pltpu
from jax.experimental.pallas import tpu_sc as plsc

B = 16384
F = 128

_NC = 2
_NS = 16
_NW = _NC * _NS
_IDX_W = 128
_NROWS = B // _IDX_W
_ROWS_PW = _NROWS // _NW


def _sc_gather(ts_hbm, ids_hbm, out_hbm, idx_v, rows_v, sem):
    wid = lax.axis_index("s") * _NC + lax.axis_index("c")
    base = wid * _ROWS_PW
    pltpu.sync_copy(ids_hbm.at[pl.ds(base, _ROWS_PW)], idx_v)
    copies = [
        pltpu.async_copy(ts_hbm.at[idx_v.at[j]], rows_v.at[j], sem)
        for j in range(_ROWS_PW)
    ]
    for c in copies:
        c.wait()
    pltpu.sync_copy(rows_v, out_hbm.at[pl.ds(base, _ROWS_PW)])


def _gather_shifts(time_shifts, ids2d):
    mesh = plsc.VectorSubcoreMesh(core_axis_name="c", subcore_axis_name="s")
    fn = pl.kernel(
        _sc_gather,
        out_type=jax.ShapeDtypeStruct((_NROWS, _IDX_W), jnp.float32),
        mesh=mesh,
        scratch_types=[
            pltpu.VMEM((_ROWS_PW, _IDX_W), jnp.int32),
            pltpu.VMEM((_ROWS_PW, _IDX_W), jnp.float32),
            pltpu.SemaphoreType.DMA,
        ],
    )
    return fn(time_shifts, ids2d)


_R = 8192
_RC = _R // _IDX_W


def _dense_body(t_ref, sh_ref, k_ref, x0_ref, o_ref):
    s = t_ref[...] + sh_ref[...]
    st = s.T
    hkv = 0.5 * jnp.exp(k_ref[...])
    hkx0 = hkv * x0_ref[...]
    for j in range(_RC):
        col = lax.slice(st, (0, j), (F, j + 1))
        o_ref[pl.ds(j * F, F), :] = 0.5 * jnp.tanh(hkv * col - hkx0) + 0.5


def _dense(t2d, sh2d, k2, x02):
    return pl.pallas_call(
        _dense_body,
        grid=(B // _R,),
        in_specs=[
            pl.BlockSpec((_RC, _IDX_W), lambda i: (i, 0)),
            pl.BlockSpec((_RC, _IDX_W), lambda i: (i, 0)),
            pl.BlockSpec((1, F), lambda i: (0, 0)),
            pl.BlockSpec((1, F), lambda i: (0, 0)),
        ],
        out_specs=pl.BlockSpec((_R, F), lambda i: (i, 0)),
        out_shape=jax.ShapeDtypeStruct((B, F), jnp.float32),
    )(t2d, sh2d, k2, x02)


def kernel(t, participant_ids, log_k_values, x0_values, time_shifts):
    ids2d = participant_ids.astype(jnp.int32).reshape(_NROWS, _IDX_W)
    shift2d = _gather_shifts(time_shifts, ids2d)
    return _dense(
        t.reshape(_NROWS, _IDX_W),
        shift2d,
        log_k_values.reshape(1, F),
        x0_values.reshape(1, F),
    )
